# baseline (speedup 1.0000x reference)
import jax
import jax.numpy as jnp
from jax import lax
from jax.experimental import pallas as pl
from jax.experimental.pallas import tpu as pltpu
from jax.experimental.pallas import tpu_sc as plsc

_B = 4096
_L = 200
_H = 128
_PAD = 1
_EPS = 1e-12
_MAXPOS = 514
_LN = 16

_NC = 2
_NS = 16
_NW = _NC * _NS


def _lane_pick(x, idx):
    return lax.gather(
        x, idx[:, None],
        lax.GatherDimensionNumbers(
            offset_dims=(), collapsed_slice_dims=(0,), start_index_map=(0,)),
        (1,), mode=lax.GatherScatterMode.PROMISE_IN_BOUNDS)


def _allsum(v, iota):
    for k in (1, 2, 4, 8):
        v = v + _lane_pick(v, iota ^ k)
    return v


def _prefix_sum(v, iota):
    for k in (1, 2, 4, 8):
        shifted = _lane_pick(v, jnp.maximum(iota - k, 0))
        v = v + jnp.where(iota >= k, shifted, 0)
    return v


def _rsqrt_vec(v):
    i = lax.bitcast_convert_type(v, jnp.int32)
    y = lax.bitcast_convert_type(jnp.int32(0x5F3759DF) - (i >> 1), jnp.float32)
    for _ in range(2):
        y = y * (1.5 - 0.5 * v * y * y)
    return y


def _make_kernel(b, l, h, maxpos, types, interpret=False):
    rows_per_w = b // _NW
    lp = ((l + _LN - 1) // _LN) * _LN
    ngroups = lp // _LN
    seg0 = min(lp, 128)
    seg1 = lp - seg0
    hc = h // _LN
    inv_h = 1.0 / h
    npos = min(maxpos, lp + 8)

    def body(ids_hbm, word_hbm, pos_hbm, tok_hbm, gam_hbm, bet_hbm, out_hbm,
             postab, tile0, tile1, otile, ids0, ids1, tok_v, gam_v, bet_v,
             semi0, semi1, semg0, semg1, semo):
        wid = lax.axis_index("s") * _NC + lax.axis_index("c")
        iota = lax.iota(jnp.int32, _LN)
        row0 = wid * rows_per_w

        pltpu.sync_copy(pos_hbm.at[pl.ds(0, npos)], postab)
        pltpu.sync_copy(tok_hbm, tok_v)
        pltpu.sync_copy(gam_hbm, gam_v)
        pltpu.sync_copy(bet_hbm, bet_v)
        tk = [tok_v[0, pl.ds(_LN * c, _LN)] for c in range(hc)]
        gam = [gam_v[pl.ds(_LN * c, _LN)] for c in range(hc)]
        bet = [bet_v[pl.ds(_LN * c, _LN)] for c in range(hc)]

        def add_tok(r, carry):
            for c in range(hc):
                postab[r, pl.ds(_LN * c, _LN)] = postab[r, pl.ds(_LN * c, _LN)] + tk[c]
            return carry

        lax.fori_loop(0, npos, add_tok, 0)

        def fire_ids(r, idb, sem):
            return pltpu.async_copy(
                ids_hbm.at[pl.ds((row0 + r) * l, l)], idb.at[pl.ds(0, l)], sem)

        def wait_ids(idb, sem):
            pltpu.make_async_copy(
                ids_hbm.at[pl.ds(0, l)], idb.at[pl.ds(0, l)], sem).wait()

        def sanitize(idb):
            if lp > l:
                tv = idb[pl.ds(lp - _LN, _LN)]
                filler = wid * _LN + iota
                tv = jnp.where(iota < _LN - (lp - l), tv, filler)
                idb[pl.ds(lp - _LN, _LN)] = tv

        def fire_gather(idb, tl, sem):
            pltpu.async_copy(
                word_hbm.at[idb.at[pl.ds(0, seg0)]], tl.at[pl.ds(0, seg0)], sem)
            if seg1:
                pltpu.async_copy(
                    word_hbm.at[idb.at[pl.ds(seg0, seg1)]],
                    tl.at[pl.ds(seg0, seg1)], sem)

        def wait_gather(idb, tl, sem):
            pltpu.make_async_copy(
                word_hbm.at[idb.at[pl.ds(0, seg0)]], tl.at[pl.ds(0, seg0)], sem).wait()
            if seg1:
                pltpu.make_async_copy(
                    word_hbm.at[idb.at[pl.ds(seg0, seg1)]],
                    tl.at[pl.ds(seg0, seg1)], sem).wait()

        def fire_out(r, tl, sem):
            pltpu.async_copy(
                tl.at[pl.ds(0, l)], out_hbm.at[pl.ds((row0 + r) * l, l)], sem)

        def wait_out(tl, sem):
            pltpu.make_async_copy(
                tl.at[pl.ds(0, l)], out_hbm.at[pl.ds(0, l)], sem).wait()

        def compute(tl, idb):
            def do_group(g, carry):
                idvec = idb[pl.ds(g * _LN, _LN)]
                maskb = idvec != _PAD
                maskv = jnp.where(maskb, 1, 0)
                pref = _prefix_sum(maskv, iota) + carry
                posv = jnp.where(maskb, pref, 0) + _PAD
                carry_out = _lane_pick(pref, jnp.full((_LN,), _LN - 1, jnp.int32))

                s1v = jnp.zeros((_LN,), jnp.float32)
                s2v = jnp.zeros((_LN,), jnp.float32)
                for t in range(_LN):
                    pos_t = posv[t]
                    tok = g * _LN + t
                    accs = [jnp.zeros((_LN,), jnp.float32) for _ in range(2)]
                    acc2s = [jnp.zeros((_LN,), jnp.float32) for _ in range(2)]
                    for c in range(hc):
                        x = (tl[tok, pl.ds(_LN * c, _LN)]
                             + postab[pos_t, pl.ds(_LN * c, _LN)])
                        otile[tok, pl.ds(_LN * c, _LN)] = x
                        accs[c % 2] = accs[c % 2] + x
                        acc2s[c % 2] = acc2s[c % 2] + x * x
                    s1 = _allsum(accs[0] + accs[1], iota)
                    s2 = _allsum(acc2s[0] + acc2s[1], iota)
                    s1v = jnp.where(iota == t, s1, s1v)
                    s2v = jnp.where(iota == t, s2, s2v)
                mu_v = s1v * inv_h
                var_v = s2v * inv_h - mu_v * mu_v
                rs_v = _rsqrt_vec(var_v + _EPS)
                for t in range(_LN):
                    tok = g * _LN + t
                    tsel = jnp.full((_LN,), t, jnp.int32)
                    mu_t = _lane_pick(mu_v, tsel)
                    rs_t = _lane_pick(rs_v, tsel)
                    for c in range(hc):
                        x = otile[tok, pl.ds(_LN * c, _LN)]
                        otile[tok, pl.ds(_LN * c, _LN)] = (
                            ((x - mu_t) * rs_t) * gam[c] + bet[c])
                return carry_out

            lax.fori_loop(0, ngroups, do_group, jnp.zeros((_LN,), jnp.int32))

        bufs = ((tile0, ids0, semi0, semg0),
                (tile1, ids1, semi1, semg1))

        fire_ids(0, ids0, semi0).wait()
        sanitize(ids0)
        fire_gather(ids0, tile0, semg0)
        fire_ids(1, ids1, semi1)

        def pair(i, carry):
            for b in range(2):
                tl, idb, si, sg = bufs[b]
                tlq, idq, siq, sgq = bufs[1 - b]
                r = 2 * i + b

                @pl.when(r < rows_per_w - 1)
                def _():
                    wait_ids(idq, siq)
                    sanitize(idq)
                    fire_gather(idq, tlq, sgq)

                wait_gather(idb, tl, sg)

                @pl.when(r >= 1)
                def _():
                    wait_out(otile, semo)

                compute(tl, idb)
                fire_out(r, otile, semo)

                @pl.when(r < rows_per_w - 2)
                def _():
                    fire_ids(r + 2, idb, si)
            return carry

        lax.fori_loop(0, rows_per_w // 2, pair, 0)
        wait_out(otile, semo)

    return pl.kernel(
        body,
        out_type=jax.ShapeDtypeStruct((b * l, h), jnp.float32),
        mesh=plsc.VectorSubcoreMesh(
            core_axis_name="c", subcore_axis_name="s",
            num_cores=_NC, num_subcores=_NS),
        scratch_types=[
            pltpu.VMEM((npos, h), jnp.float32),
            pltpu.VMEM((lp, h), jnp.float32),
            pltpu.VMEM((lp, h), jnp.float32),
            pltpu.VMEM((lp, h), jnp.float32),
            pltpu.VMEM((lp,), jnp.int32),
            pltpu.VMEM((lp,), jnp.int32),
            pltpu.VMEM((types, h), jnp.float32),
            pltpu.VMEM((h,), jnp.float32),
            pltpu.VMEM((h,), jnp.float32),
            pltpu.SemaphoreType.DMA,
            pltpu.SemaphoreType.DMA,
            pltpu.SemaphoreType.DMA,
            pltpu.SemaphoreType.DMA,
            pltpu.SemaphoreType.DMA,
        ],
        interpret=interpret,
    )


def kernel(input_ids, word_emb, pos_emb, tok_emb, ln_gamma, ln_beta):
    k = _make_kernel(_B, _L, _H, _MAXPOS, tok_emb.shape[0])
    out = k(input_ids.reshape(-1), word_emb, pos_emb, tok_emb, ln_gamma, ln_beta)
    return out.reshape(_B, _L, _H)


# speedup vs baseline: 1.6501x; 1.6501x over previous
import jax
import jax.numpy as jnp
from jax import lax
from jax.experimental import pallas as pl
from jax.experimental.pallas import tpu as pltpu
from jax.experimental.pallas import tpu_sc as plsc

_B = 4096
_L = 200
_H = 128
_PAD = 1
_EPS = 1e-12
_MAXPOS = 514
_LN = 16

_NC = 2
_NS = 16
_NW = _NC * _NS


def _lane_pick(x, idx):
    return lax.gather(
        x, idx[:, None],
        lax.GatherDimensionNumbers(
            offset_dims=(), collapsed_slice_dims=(0,), start_index_map=(0,)),
        (1,), mode=lax.GatherScatterMode.PROMISE_IN_BOUNDS)


def _allsum(v, iota):
    for k in (1, 2, 4, 8):
        v = v + _lane_pick(v, iota ^ k)
    return v


def _prefix_sum(v, iota):
    for k in (1, 2, 4, 8):
        shifted = _lane_pick(v, jnp.maximum(iota - k, 0))
        v = v + jnp.where(iota >= k, shifted, 0)
    return v


def _rsqrt_vec(v):
    i = lax.bitcast_convert_type(v, jnp.int32)
    y = lax.bitcast_convert_type(jnp.int32(0x5F3759DF) - (i >> 1), jnp.float32)
    for _ in range(2):
        y = y * (1.5 - 0.5 * v * y * y)
    return y


def _make_kernel(b, l, h, maxpos, types, interpret=False):
    rows_per_w = b // _NW
    lp = ((l + _LN - 1) // _LN) * _LN
    ngroups = lp // _LN
    seg0 = min(lp, 128)
    seg1 = lp - seg0
    hc = h // _LN
    inv_h = 1.0 / h
    npos = min(maxpos, lp + 8)

    def body(ids_hbm, word_hbm, pos_hbm, tok_hbm, gam_hbm, bet_hbm, out_hbm,
             postab, tile0, tile1, otile, ids0, ids1, tok_v, gam_v, bet_v,
             semi0, semi1, semg0, semg1, semo):
        wid = lax.axis_index("s") * _NC + lax.axis_index("c")
        iota = lax.iota(jnp.int32, _LN)
        row0 = wid * rows_per_w

        pltpu.sync_copy(pos_hbm.at[pl.ds(0, npos)], postab)
        pltpu.sync_copy(tok_hbm, tok_v)
        pltpu.sync_copy(gam_hbm, gam_v)
        pltpu.sync_copy(bet_hbm, bet_v)
        tk = [tok_v[0, pl.ds(_LN * c, _LN)] for c in range(hc)]
        gam = [gam_v[pl.ds(_LN * c, _LN)] for c in range(hc)]
        bet = [bet_v[pl.ds(_LN * c, _LN)] for c in range(hc)]

        def add_tok(r, carry):
            for c in range(hc):
                postab[r, pl.ds(_LN * c, _LN)] = postab[r, pl.ds(_LN * c, _LN)] + tk[c]
            return carry

        lax.fori_loop(0, npos, add_tok, 0)

        def fire_ids(r, idb, sem):
            return pltpu.async_copy(
                ids_hbm.at[pl.ds((row0 + r) * l, l)], idb.at[pl.ds(0, l)], sem)

        def wait_ids(idb, sem):
            pltpu.make_async_copy(
                ids_hbm.at[pl.ds(0, l)], idb.at[pl.ds(0, l)], sem).wait()

        def sanitize(idb):
            if lp > l:
                tv = idb[pl.ds(lp - _LN, _LN)]
                filler = wid * _LN + iota
                tv = jnp.where(iota < _LN - (lp - l), tv, filler)
                idb[pl.ds(lp - _LN, _LN)] = tv

        def fire_gather(idb, tl, sem):
            pltpu.async_copy(
                word_hbm.at[idb.at[pl.ds(0, seg0)]], tl.at[pl.ds(0, seg0)], sem)
            if seg1:
                pltpu.async_copy(
                    word_hbm.at[idb.at[pl.ds(seg0, seg1)]],
                    tl.at[pl.ds(seg0, seg1)], sem)

        def wait_gather(idb, tl, sem):
            pltpu.make_async_copy(
                word_hbm.at[idb.at[pl.ds(0, seg0)]], tl.at[pl.ds(0, seg0)], sem).wait()
            if seg1:
                pltpu.make_async_copy(
                    word_hbm.at[idb.at[pl.ds(seg0, seg1)]],
                    tl.at[pl.ds(seg0, seg1)], sem).wait()

        def fire_out(r, tl, sem):
            pltpu.async_copy(
                tl.at[pl.ds(0, l)], out_hbm.at[pl.ds((row0 + r) * l, l)], sem)

        def wait_out(tl, sem):
            pltpu.make_async_copy(
                tl.at[pl.ds(0, l)], out_hbm.at[pl.ds(0, l)], sem).wait()

        def compute(tl, idb):
            def do_group(g, carry):
                idvec = idb[pl.ds(g * _LN, _LN)]
                maskb = idvec != _PAD
                maskv = jnp.where(maskb, 1, 0)
                pref = _prefix_sum(maskv, iota) + carry
                posv = jnp.where(maskb, pref, 0) + _PAD
                carry_out = _lane_pick(pref, jnp.full((_LN,), _LN - 1, jnp.int32))

                for tp in range(_LN // 2):
                    toks = (g * _LN + 2 * tp, g * _LN + 2 * tp + 1)
                    poss = (posv[2 * tp], posv[2 * tp + 1])
                    xs = [[], []]
                    acc = [jnp.zeros((_LN,), jnp.float32) for _ in range(2)]
                    acc2 = [jnp.zeros((_LN,), jnp.float32) for _ in range(2)]
                    for c in range(hc):
                        for j in range(2):
                            x = (tl[toks[j], pl.ds(_LN * c, _LN)]
                                 + postab[poss[j], pl.ds(_LN * c, _LN)])
                            xs[j].append(x)
                            acc[j] = acc[j] + x
                            acc2[j] = acc2[j] + x * x
                    mus, rss = [], []
                    for j in range(2):
                        s1 = _allsum(acc[j], iota)
                        s2 = _allsum(acc2[j], iota)
                        mu = s1 * inv_h
                        var = s2 * inv_h - mu * mu
                        mus.append(mu)
                        rss.append(_rsqrt_vec(var + _EPS))
                    for c in range(hc):
                        for j in range(2):
                            otile[toks[j], pl.ds(_LN * c, _LN)] = (
                                ((xs[j][c] - mus[j]) * rss[j]) * gam[c] + bet[c])
                return carry_out

            lax.fori_loop(0, ngroups, do_group, jnp.zeros((_LN,), jnp.int32))

        bufs = ((tile0, ids0, semi0, semg0),
                (tile1, ids1, semi1, semg1))

        fire_ids(0, ids0, semi0).wait()
        sanitize(ids0)
        fire_gather(ids0, tile0, semg0)
        fire_ids(1, ids1, semi1)

        def pair(i, carry):
            for b in range(2):
                tl, idb, si, sg = bufs[b]
                tlq, idq, siq, sgq = bufs[1 - b]
                r = 2 * i + b

                @pl.when(r < rows_per_w - 1)
                def _():
                    wait_ids(idq, siq)
                    sanitize(idq)
                    fire_gather(idq, tlq, sgq)

                wait_gather(idb, tl, sg)

                @pl.when(r >= 1)
                def _():
                    wait_out(otile, semo)

                compute(tl, idb)
                fire_out(r, otile, semo)

                @pl.when(r < rows_per_w - 2)
                def _():
                    fire_ids(r + 2, idb, si)
            return carry

        lax.fori_loop(0, rows_per_w // 2, pair, 0)
        wait_out(otile, semo)

    return pl.kernel(
        body,
        out_type=jax.ShapeDtypeStruct((b * l, h), jnp.float32),
        mesh=plsc.VectorSubcoreMesh(
            core_axis_name="c", subcore_axis_name="s",
            num_cores=_NC, num_subcores=_NS),
        scratch_types=[
            pltpu.VMEM((npos, h), jnp.float32),
            pltpu.VMEM((lp, h), jnp.float32),
            pltpu.VMEM((lp, h), jnp.float32),
            pltpu.VMEM((lp, h), jnp.float32),
            pltpu.VMEM((lp,), jnp.int32),
            pltpu.VMEM((lp,), jnp.int32),
            pltpu.VMEM((types, h), jnp.float32),
            pltpu.VMEM((h,), jnp.float32),
            pltpu.VMEM((h,), jnp.float32),
            pltpu.SemaphoreType.DMA,
            pltpu.SemaphoreType.DMA,
            pltpu.SemaphoreType.DMA,
            pltpu.SemaphoreType.DMA,
            pltpu.SemaphoreType.DMA,
        ],
        interpret=interpret,
    )


def kernel(input_ids, word_emb, pos_emb, tok_emb, ln_gamma, ln_beta):
    k = _make_kernel(_B, _L, _H, _MAXPOS, tok_emb.shape[0])
    out = k(input_ids.reshape(-1), word_emb, pos_emb, tok_emb, ln_gamma, ln_beta)
    return out.reshape(_B, _L, _H)
